# baseline (speedup 1.0000x reference)
import functools

import jax
import jax.numpy as jnp
from jax import lax
from jax.experimental import pallas as pl
from jax.experimental.pallas import tpu as pltpu
from jax.experimental.pallas import tpu_sc as plsc

N = 50000
E = 800000
D = 100

NC = 2
NS = 16
L = 16
DP = 112
NB = 6
BR = 8344
NP = NB * BR
BPC = NB // NC
R = 2000
EPT = E // NS
NR = EPT // R
K = 112
NBUF = 4
T = 1536
BCAP = T + R + K + 8
ACC_STRIPE = 528
ACC_ROWS = ACC_STRIPE * NS
DUMMY_ROW = BR


def _layer_body(row_hbm, col_hbm, w_hbm, table_hbm, out_hbm,
                e_row, e_col, e_w, b_col, b_w, b_rl,
                idx0, idx1, idx2, idx3,
                gbuf0, gbuf1, gbuf2, gbuf3, acc,
                gs0, gs1, gs2, gs3, ss0, ss1, ss2, ss3, sem_st):
  c = lax.axis_index("c")
  s = lax.axis_index("s")
  ebase = s * EPT
  ziota = lax.iota(jnp.int32, L)
  gbufs = (gbuf0, gbuf1, gbuf2, gbuf3)
  idxs = (idx0, idx1, idx2, idx3)
  gsems = (gs0, gs1, gs2, gs3)
  ssems = (ss0, ss1, ss2, ss3)

  def start_gather(j, b):
    pltpu.make_async_copy(
        table_hbm.at[b_col.at[pl.ds(j * K, K)]], gbufs[b], gsems[b]).start()

  def wait_gather(b):
    pltpu.make_async_copy(
        table_hbm.at[b_col.at[pl.ds(0, K)]], gbufs[b], gsems[b]).wait()

  def wait_scatter(b):
    pltpu.make_async_copy(gbufs[b], acc.at[idxs[b]], ssems[b]).wait()

  def scale_scatter(j, b):
    gb = gbufs[b]
    koff = j * K
    for q in range(K // L):
      idxs[b][pl.ds(q * L, L)] = b_rl[pl.ds(koff + q * L, L)]

    @plsc.parallel_loop(0, K, unroll=8)
    def _(e2):
      wv = plsc.load_gather(
          b_w, [jnp.zeros((L,), jnp.int32) + (koff + e2)])
      for q in range(DP // L):
        gb[e2, pl.ds(q * L, L)] = gb[e2, pl.ds(q * L, L)] * wv

    pltpu.make_async_copy(gb, acc.at[idxs[b]], ssems[b]).start(add=True)

  def process_chunks(nch):
    @pl.when(nch > 0)
    def _():
      start_gather(0, 0)

    @pl.when(nch > 1)
    def _():
      start_gather(1, 1)

    def pipe(jj, _):
      jbase = jj * NBUF
      for b in range(NBUF):
        j = jbase + b
        jr = j + 2
        br = (b + 2) % NBUF

        @pl.when(j < nch)
        def _(j=j, b=b):
          wait_gather(b)
          scale_scatter(j, b)

        @pl.when(jr < nch)
        def _(jr=jr, br=br):
          @pl.when(jr >= NBUF)
          def _():
            wait_scatter(br)
          start_gather(jr, br)
      return 0
    lax.fori_loop(0, (nch + (NBUF - 1)) // NBUF, pipe, 0)

    for b in range(NBUF):
      @pl.when(nch > b)
      def _(b=b):
        wait_scatter(b)

  for blk in range(BPC):
    lo = (c * BPC + blk) * BR

    def zrow(r, _):
      for q in range(DP // L):
        gbuf0[r, pl.ds(q * L, L)] = jnp.zeros((L,), jnp.float32)
      return 0
    lax.fori_loop(0, K, zrow, 0)
    for q in range(ACC_STRIPE // K):
      pltpu.sync_copy(gbuf0, acc.at[pl.ds(s * ACC_STRIPE + q * K, K)])
    if ACC_STRIPE % K:
      pltpu.sync_copy(
          gbuf0.at[pl.ds(0, ACC_STRIPE % K)],
          acc.at[pl.ds(s * ACC_STRIPE + (ACC_STRIPE // K) * K,
                       ACC_STRIPE % K)])
    plsc.subcore_barrier()

    def stage_start(r):
      off = ebase + r * R
      pltpu.make_async_copy(row_hbm.at[pl.ds(off, R)], e_row, sem_st).start()
      pltpu.make_async_copy(col_hbm.at[pl.ds(off, R)], e_col, sem_st).start()
      pltpu.make_async_copy(w_hbm.at[pl.ds(off, R)], e_w, sem_st).start()

    def stage_wait():
      pltpu.make_async_copy(row_hbm.at[pl.ds(0, R)], e_row, sem_st).wait()
      pltpu.make_async_copy(col_hbm.at[pl.ds(0, R)], e_col, sem_st).wait()
      pltpu.make_async_copy(w_hbm.at[pl.ds(0, R)], e_w, sem_st).wait()

    stage_start(0)

    def round_body(r, cnt):
      stage_wait()

      def comp(i, cnt):
        rows = e_row[pl.ds(i * L, L)]
        cols = e_col[pl.ds(i * L, L)]
        ws = e_w[pl.ds(i * L, L)]
        m = (rows >= lo) & (rows < lo + BR)
        mi = jnp.where(m, jnp.ones((L,), jnp.int32),
                       jnp.zeros((L,), jnp.int32))
        pos = cnt + plsc.cumsum(mi) - 1
        plsc.store_scatter(b_col, [pos], cols, mask=m)
        plsc.store_scatter(b_w, [pos], ws, mask=m)
        plsc.store_scatter(b_rl, [pos], rows - lo, mask=m)
        return cnt + jnp.sum(mi)
      cnt = lax.fori_loop(0, R // L, comp, cnt)

      @pl.when(r + 1 < NR)
      def _():
        stage_start(r + 1)

      nch = jnp.where(cnt >= T, cnt // K, 0)
      process_chunks(nch)
      rem_base = nch * K
      for q in range(K // L):
        b_col[pl.ds(q * L, L)] = b_col[pl.ds(rem_base + q * L, L)]
        b_w[pl.ds(q * L, L)] = b_w[pl.ds(rem_base + q * L, L)]
        b_rl[pl.ds(q * L, L)] = b_rl[pl.ds(rem_base + q * L, L)]
      return cnt - rem_base
    cnt = lax.fori_loop(0, NR, round_body, jnp.int32(0))

    for q in range(K // L):
      padpos = cnt + q * L + ziota
      plsc.store_scatter(b_col, [padpos], jnp.zeros((L,), jnp.int32))
      plsc.store_scatter(b_w, [padpos], jnp.zeros((L,), jnp.float32))
      plsc.store_scatter(b_rl, [padpos],
                         jnp.full((L,), DUMMY_ROW, jnp.int32))
    process_chunks((cnt + (K - 1)) // K)
    plsc.subcore_barrier()

    last = BR - (NS - 1) * ACC_STRIPE

    @pl.when(s < NS - 1)
    def _():
      pltpu.sync_copy(acc.at[pl.ds(s * ACC_STRIPE, ACC_STRIPE)],
                      out_hbm.at[pl.ds(lo + s * ACC_STRIPE, ACC_STRIPE)])

    @pl.when(s == NS - 1)
    def _():
      pltpu.sync_copy(acc.at[pl.ds((NS - 1) * ACC_STRIPE, last)],
                      out_hbm.at[pl.ds(lo + (NS - 1) * ACC_STRIPE, last)])


_sc_layer = pl.kernel(
    _layer_body,
    out_type=jax.ShapeDtypeStruct((NP, DP), jnp.float32),
    mesh=plsc.VectorSubcoreMesh(core_axis_name="c", subcore_axis_name="s",
                                num_cores=NC, num_subcores=NS),
    compiler_params=pltpu.CompilerParams(needs_layout_passes=False,
                                         use_tc_tiling_on_sc=False),
    scratch_types=[
        pltpu.VMEM((R,), jnp.int32),
        pltpu.VMEM((R,), jnp.int32),
        pltpu.VMEM((R,), jnp.float32),
        pltpu.VMEM((BCAP,), jnp.int32),
        pltpu.VMEM((BCAP,), jnp.float32),
        pltpu.VMEM((BCAP,), jnp.int32),
        pltpu.VMEM((K,), jnp.int32),
        pltpu.VMEM((K,), jnp.int32),
        pltpu.VMEM((K,), jnp.int32),
        pltpu.VMEM((K,), jnp.int32),
        pltpu.VMEM((K, DP), jnp.float32),
        pltpu.VMEM((K, DP), jnp.float32),
        pltpu.VMEM((K, DP), jnp.float32),
        pltpu.VMEM((K, DP), jnp.float32),
        pltpu.VMEM_SHARED((ACC_ROWS, DP), jnp.float32),
        pltpu.SemaphoreType.DMA,
        pltpu.SemaphoreType.DMA,
        pltpu.SemaphoreType.DMA,
        pltpu.SemaphoreType.DMA,
        pltpu.SemaphoreType.DMA,
        pltpu.SemaphoreType.DMA,
        pltpu.SemaphoreType.DMA,
        pltpu.SemaphoreType.DMA,
        pltpu.SemaphoreType.DMA,
    ],
)


_CROWS = BR


def _combine_body(a_ref, h0, h1, h2, h3, o_ref):
  acc = jnp.zeros((_CROWS, DP), jnp.float32)
  for l, h in enumerate((h0, h1, h2, h3)):
    x = h[...]
    ss = jnp.sum(x * x, axis=-1, keepdims=True)
    nrm = jnp.maximum(jnp.sqrt(ss), 1e-12)
    acc = acc + a_ref[l] * (x / nrm)
  o_ref[...] = acc


_combine = pl.pallas_call(
    _combine_body,
    grid=(NP // _CROWS,),
    in_specs=[
        pl.BlockSpec(memory_space=pltpu.SMEM),
    ] + [pl.BlockSpec((_CROWS, DP), lambda i: (i, 0)) for _ in range(4)],
    out_specs=pl.BlockSpec((_CROWS, DP), lambda i: (i, 0)),
    out_shape=jax.ShapeDtypeStruct((NP, DP), jnp.float32),
)


def kernel(edge_index, edge_weight, embedding, a):
  row = edge_index[0]
  col = edge_index[1]
  x0 = jnp.pad(embedding, ((0, NP - N), (0, DP - D)))
  h1 = _sc_layer(row, col, edge_weight, x0)
  h2 = _sc_layer(row, col, edge_weight, h1)
  h3 = _sc_layer(row, col, edge_weight, h2)
  out = _combine(a.reshape(-1), x0, h1, h2, h3)
  return out[:N, :D]


# speedup vs baseline: 1.0269x; 1.0269x over previous
import jax
import jax.numpy as jnp
from jax import lax
from jax.experimental import pallas as pl
from jax.experimental.pallas import tpu as pltpu
from jax.experimental.pallas import tpu_sc as plsc

N = 50000
E = 800000
D = 100

NC = 2
NS = 16
NW = NC * NS
L = 16
DP = 112
NB = 6
BR = 8344
NP = NB * BR
BPC = NB // NC
K = 112
NBUF = 4
ACC_STRIPE = 528
ACC_ROWS = ACC_STRIPE * NS
DUMMY_ROW = BR

EPP = 25088
E2 = NW * EPP
PR = 1568
PNR = EPP // PR
RING = 4096
FLW = 2048
CAP = 28672
PAD = 128

SB = 9 * K


def _partition_body(row_hbm, col_hbm, w_hbm,
                    lcol_hbm, lw_hbm, lrl_hbm, cnts_hbm,
                    e_row, e_col, e_w,
                    rc0, rc1, rc2, rc3, rc4, rc5,
                    rw0, rw1, rw2, rw3, rw4, rw5,
                    rl0, rl1, rl2, rl3, rl4, rl5,
                    cvec, sem_st):
  c = lax.axis_index("c")
  s = lax.axis_index("s")
  p = c * NS + s
  ebase = p * EPP
  ziota = lax.iota(jnp.int32, L)
  rcs = (rc0, rc1, rc2, rc3, rc4, rc5)
  rws = (rw0, rw1, rw2, rw3, rw4, rw5)
  rls = (rl0, rl1, rl2, rl3, rl4, rl5)

  def stage_start(r):
    off = ebase + r * PR
    pltpu.make_async_copy(row_hbm.at[pl.ds(off, PR)], e_row, sem_st).start()
    pltpu.make_async_copy(col_hbm.at[pl.ds(off, PR)], e_col, sem_st).start()
    pltpu.make_async_copy(w_hbm.at[pl.ds(off, PR)], e_w, sem_st).start()

  def stage_wait():
    pltpu.make_async_copy(row_hbm.at[pl.ds(0, PR)], e_row, sem_st).wait()
    pltpu.make_async_copy(col_hbm.at[pl.ds(0, PR)], e_col, sem_st).wait()
    pltpu.make_async_copy(w_hbm.at[pl.ds(0, PR)], e_w, sem_st).wait()

  def flush(blk, fl):
    vo = pl.multiple_of(fl & (RING - 1), FLW)
    base = pl.multiple_of((p * NB + blk) * CAP + fl, FLW)
    pltpu.sync_copy(rcs[blk].at[pl.ds(vo, FLW)], lcol_hbm.at[pl.ds(base, FLW)])
    pltpu.sync_copy(rws[blk].at[pl.ds(vo, FLW)], lw_hbm.at[pl.ds(base, FLW)])
    pltpu.sync_copy(rls[blk].at[pl.ds(vo, FLW)], lrl_hbm.at[pl.ds(base, FLW)])

  def round_body(r, state):
    stage_wait()
    cnts = state[:NB]
    fls = state[NB:]

    def comp(i, cnts):
      rows = e_row[pl.ds(i * L, L)]
      cols = e_col[pl.ds(i * L, L)]
      ws = e_w[pl.ds(i * L, L)]
      new = []
      for blk in range(NB):
        lo = blk * BR
        m = (rows >= lo) & (rows < lo + BR)
        mi = jnp.where(m, jnp.ones((L,), jnp.int32),
                       jnp.zeros((L,), jnp.int32))
        pos = (cnts[blk] + plsc.cumsum(mi) - 1) & (RING - 1)
        plsc.store_scatter(rcs[blk], [pos], cols, mask=m)
        plsc.store_scatter(rws[blk], [pos], ws, mask=m)
        plsc.store_scatter(rls[blk], [pos], rows - lo, mask=m)
        new.append(cnts[blk] + jnp.sum(mi))
      return tuple(new)
    cnts = lax.fori_loop(0, PR // L, comp, cnts)

    @pl.when(r + 1 < PNR)
    def _():
      stage_start(r + 1)

    new_fls = []
    for blk in range(NB):
      pending = cnts[blk] - fls[blk]
      do = pending >= FLW

      @pl.when(do)
      def _(blk=blk, fl=fls[blk]):
        flush(blk, fl)
      new_fls.append(jnp.where(do, fls[blk] + FLW, fls[blk]))
    return cnts + tuple(new_fls)

  stage_start(0)
  z = jnp.int32(0)
  state = lax.fori_loop(0, PNR, round_body, (z,) * (2 * NB))
  cnts = state[:NB]
  fls = state[NB:]

  cv = jnp.zeros((L,), jnp.int32)
  for blk in range(NB):
    for q in range(PAD // L):
      padpos = (cnts[blk] + q * L + ziota) & (RING - 1)
      plsc.store_scatter(rcs[blk], [padpos], jnp.zeros((L,), jnp.int32))
      plsc.store_scatter(rws[blk], [padpos], jnp.zeros((L,), jnp.float32))
      plsc.store_scatter(rls[blk], [padpos],
                         jnp.full((L,), DUMMY_ROW, jnp.int32))
    pending = cnts[blk] + PAD - fls[blk]

    @pl.when(pending > 0)
    def _(blk=blk, fl=fls[blk]):
      flush(blk, fl)

    @pl.when(pending > FLW)
    def _(blk=blk, fl=fls[blk] + FLW):
      flush(blk, fl)
    cv = jnp.where(ziota == blk, jnp.zeros((L,), jnp.int32) + cnts[blk], cv)
  cvec[pl.ds(0, L)] = cv
  pltpu.sync_copy(cvec, cnts_hbm.at[pl.ds(p * L, L)])


_sc_partition = pl.kernel(
    _partition_body,
    out_type=(
        jax.ShapeDtypeStruct((NW * NB * CAP,), jnp.int32),
        jax.ShapeDtypeStruct((NW * NB * CAP,), jnp.float32),
        jax.ShapeDtypeStruct((NW * NB * CAP,), jnp.int32),
        jax.ShapeDtypeStruct((NW * L,), jnp.int32),
    ),
    mesh=plsc.VectorSubcoreMesh(core_axis_name="c", subcore_axis_name="s",
                                num_cores=NC, num_subcores=NS),
    compiler_params=pltpu.CompilerParams(needs_layout_passes=False,
                                         use_tc_tiling_on_sc=False),
    scratch_types=[
        pltpu.VMEM((PR,), jnp.int32),
        pltpu.VMEM((PR,), jnp.int32),
        pltpu.VMEM((PR,), jnp.float32),
    ] + [pltpu.VMEM((RING,), jnp.int32) for _ in range(NB)]
    + [pltpu.VMEM((RING,), jnp.float32) for _ in range(NB)]
    + [pltpu.VMEM((RING,), jnp.int32) for _ in range(NB)]
    + [
        pltpu.VMEM((L,), jnp.int32),
        pltpu.SemaphoreType.DMA,
    ],
)


def _layer_body(lcol_hbm, lw_hbm, lrl_hbm, cnts_hbm, table_hbm, out_hbm,
                b_col, b_w, b_rl, cvec,
                idx0, idx1, idx2, idx3,
                gbuf0, gbuf1, gbuf2, gbuf3, acc,
                gs0, gs1, gs2, gs3, ss0, ss1, ss2, ss3, sem_st):
  c = lax.axis_index("c")
  s = lax.axis_index("s")
  ziota = lax.iota(jnp.int32, L)
  gbufs = (gbuf0, gbuf1, gbuf2, gbuf3)
  idxs = (idx0, idx1, idx2, idx3)
  gsems = (gs0, gs1, gs2, gs3)
  ssems = (ss0, ss1, ss2, ss3)

  cnts = []
  for pp in range(2):
    pltpu.sync_copy(cnts_hbm.at[pl.ds((2 * s + pp) * L, L)], cvec)
    cv = cvec[pl.ds(0, L)]
    row = []
    for blk in range(BPC):
      bg = c * BPC + blk
      row.append(jnp.sum(jnp.where(ziota == bg,
                                   cv, jnp.zeros((L,), jnp.int32))))
    cnts.append(row)

  def start_gather(koff, b):
    koff = pl.multiple_of(koff, L)
    pltpu.make_async_copy(
        table_hbm.at[b_col.at[pl.ds(koff, K)]], gbufs[b], gsems[b]).start()

  def wait_gather(b):
    pltpu.make_async_copy(
        table_hbm.at[b_col.at[pl.ds(0, K)]], gbufs[b], gsems[b]).wait()

  def wait_scatter(b):
    pltpu.make_async_copy(gbufs[b], acc.at[idxs[b]], ssems[b]).wait()

  def scale_scatter(koff, b):
    gb = gbufs[b]
    koff = pl.multiple_of(koff, L)
    for q in range(K // L):
      idxs[b][pl.ds(q * L, L)] = b_rl[pl.ds(koff + q * L, L)]

    @plsc.parallel_loop(0, K, unroll=4)
    def _(e2):
      wv = plsc.load_gather(
          b_w, [jnp.zeros((L,), jnp.int32) + (koff + e2)])
      for q in range(DP // L):
        gb[e2, pl.ds(q * L, L)] = gb[e2, pl.ds(q * L, L)] * wv

    pltpu.make_async_copy(gb, acc.at[idxs[b]], ssems[b]).start(add=True)

  def process_chunks(vbase, nch):
    @pl.when(nch > 0)
    def _():
      start_gather(vbase, 0)

    @pl.when(nch > 1)
    def _():
      start_gather(vbase + K, 1)

    def pipe(jj, _):
      jbase = jj * NBUF
      for b in range(NBUF):
        j = jbase + b
        jr = j + 2
        br = (b + 2) % NBUF

        @pl.when(j < nch)
        def _(j=j, b=b):
          wait_gather(b)
          scale_scatter(vbase + j * K, b)

        @pl.when(jr < nch)
        def _(jr=jr, br=br):
          @pl.when(jr >= NBUF)
          def _():
            wait_scatter(br)
          start_gather(vbase + jr * K, br)
      return 0
    lax.fori_loop(0, (nch + (NBUF - 1)) // NBUF, pipe, 0)

    for b in range(NBUF):
      @pl.when(nch > b)
      def _(b=b):
        wait_scatter(b)

  def sb_start(hbase, sb):
    off = pl.multiple_of(hbase + sb * SB, L)
    vo = pl.multiple_of((sb % 2) * SB, L)
    pltpu.make_async_copy(lcol_hbm.at[pl.ds(off, SB)],
                          b_col.at[pl.ds(vo, SB)], sem_st).start()
    pltpu.make_async_copy(lw_hbm.at[pl.ds(off, SB)],
                          b_w.at[pl.ds(vo, SB)], sem_st).start()
    pltpu.make_async_copy(lrl_hbm.at[pl.ds(off, SB)],
                          b_rl.at[pl.ds(vo, SB)], sem_st).start()

  def sb_wait():
    pltpu.make_async_copy(lcol_hbm.at[pl.ds(0, SB)],
                          b_col.at[pl.ds(0, SB)], sem_st).wait()
    pltpu.make_async_copy(lw_hbm.at[pl.ds(0, SB)],
                          b_w.at[pl.ds(0, SB)], sem_st).wait()
    pltpu.make_async_copy(lrl_hbm.at[pl.ds(0, SB)],
                          b_rl.at[pl.ds(0, SB)], sem_st).wait()

  for blk in range(BPC):
    bg = c * BPC + blk
    lo = bg * BR

    def zrow(r, _):
      for q in range(DP // L):
        gbuf0[r, pl.ds(q * L, L)] = jnp.zeros((L,), jnp.float32)
      return 0
    lax.fori_loop(0, K, zrow, 0)
    for q in range(ACC_STRIPE // K):
      pltpu.sync_copy(gbuf0, acc.at[pl.ds(s * ACC_STRIPE + q * K, K)])
    if ACC_STRIPE % K:
      pltpu.sync_copy(
          gbuf0.at[pl.ds(0, ACC_STRIPE % K)],
          acc.at[pl.ds(s * ACC_STRIPE + (ACC_STRIPE // K) * K,
                       ACC_STRIPE % K)])
    plsc.subcore_barrier()

    for pp in range(2):
      cnt = cnts[pp][blk]
      hbase = ((2 * s + pp) * NB + bg) * CAP
      nch = (cnt + (K - 1)) // K
      nsb = (nch + 8) // 9

      @pl.when(nsb > 0)
      def _(hbase=hbase):
        sb_start(hbase, 0)

      def sb_body(sb, _, hbase=hbase, nch=nch, nsb=nsb):
        sb_wait()

        @pl.when(sb + 1 < nsb)
        def _():
          sb_start(hbase, sb + 1)
        nloc = jnp.minimum(nch - sb * 9, 9)
        process_chunks((sb % 2) * SB, nloc)
        return 0
      lax.fori_loop(0, nsb, sb_body, 0)
    plsc.subcore_barrier()

    last = BR - (NS - 1) * ACC_STRIPE

    @pl.when(s < NS - 1)
    def _():
      pltpu.sync_copy(acc.at[pl.ds(s * ACC_STRIPE, ACC_STRIPE)],
                      out_hbm.at[pl.ds(lo + s * ACC_STRIPE, ACC_STRIPE)])

    @pl.when(s == NS - 1)
    def _():
      pltpu.sync_copy(acc.at[pl.ds((NS - 1) * ACC_STRIPE, last)],
                      out_hbm.at[pl.ds(lo + (NS - 1) * ACC_STRIPE, last)])


_sc_layer = pl.kernel(
    _layer_body,
    out_type=jax.ShapeDtypeStruct((NP, DP), jnp.float32),
    mesh=plsc.VectorSubcoreMesh(core_axis_name="c", subcore_axis_name="s",
                                num_cores=NC, num_subcores=NS),
    compiler_params=pltpu.CompilerParams(needs_layout_passes=False,
                                         use_tc_tiling_on_sc=False),
    scratch_types=[
        pltpu.VMEM((2 * SB,), jnp.int32),
        pltpu.VMEM((2 * SB,), jnp.float32),
        pltpu.VMEM((2 * SB,), jnp.int32),
        pltpu.VMEM((L,), jnp.int32),
        pltpu.VMEM((K,), jnp.int32),
        pltpu.VMEM((K,), jnp.int32),
        pltpu.VMEM((K,), jnp.int32),
        pltpu.VMEM((K,), jnp.int32),
        pltpu.VMEM((K, DP), jnp.float32),
        pltpu.VMEM((K, DP), jnp.float32),
        pltpu.VMEM((K, DP), jnp.float32),
        pltpu.VMEM((K, DP), jnp.float32),
        pltpu.VMEM_SHARED((ACC_ROWS, DP), jnp.float32),
        pltpu.SemaphoreType.DMA,
        pltpu.SemaphoreType.DMA,
        pltpu.SemaphoreType.DMA,
        pltpu.SemaphoreType.DMA,
        pltpu.SemaphoreType.DMA,
        pltpu.SemaphoreType.DMA,
        pltpu.SemaphoreType.DMA,
        pltpu.SemaphoreType.DMA,
        pltpu.SemaphoreType.DMA,
    ],
)


_CROWS = BR


def _combine_body(a_ref, h0, h1, h2, h3, o_ref):
  acc = jnp.zeros((_CROWS, DP), jnp.float32)
  for l, h in enumerate((h0, h1, h2, h3)):
    x = h[...]
    ss = jnp.sum(x * x, axis=-1, keepdims=True)
    nrm = jnp.maximum(jnp.sqrt(ss), 1e-12)
    acc = acc + a_ref[l] * (x / nrm)
  o_ref[...] = acc


_combine = pl.pallas_call(
    _combine_body,
    grid=(NP // _CROWS,),
    in_specs=[
        pl.BlockSpec(memory_space=pltpu.SMEM),
    ] + [pl.BlockSpec((_CROWS, DP), lambda i: (i, 0)) for _ in range(4)],
    out_specs=pl.BlockSpec((_CROWS, DP), lambda i: (i, 0)),
    out_shape=jax.ShapeDtypeStruct((NP, DP), jnp.float32),
)


def kernel(edge_index, edge_weight, embedding, a):
  row = jnp.pad(edge_index[0], (0, E2 - E), constant_values=NP)
  col = jnp.pad(edge_index[1], (0, E2 - E))
  w = jnp.pad(edge_weight, (0, E2 - E))
  x0 = jnp.pad(embedding, ((0, NP - N), (0, DP - D)))
  lcol, lw, lrl, cnts = _sc_partition(row, col, w)
  h1 = _sc_layer(lcol, lw, lrl, cnts, x0)
  h2 = _sc_layer(lcol, lw, lrl, cnts, h1)
  h3 = _sc_layer(lcol, lw, lrl, cnts, h2)
  out = _combine(a.reshape(-1), x0, h1, h2, h3)
  return out[:N, :D]
